# baseline (speedup 1.0000x reference)
import jax
import jax.numpy as jnp
from jax import lax
from jax.experimental import pallas as pl
from jax.experimental.pallas import tpu as pltpu
from jax.experimental.pallas import tpu_sc as plsc

B = 8
V = 10000
F = 256
NNEW = 20000
NC, NS, L = 2, 16, 16
NW = NC * NS
PER_W = (B * NNEW) // NW
WPB = NW // B
N_PER_W = NNEW // WPB
COPY_W = 2504
COPY_LAST = V - COPY_W
CHUNK = 112
NCH = 46
LAST_OFF = PER_W - CHUNK
IDX_PAD = 5008
VOUT = V + NNEW


def _avg_into(dst, src, rows):
    def row_body(r, c):
        for j in range(F // L):
            sl = pl.ds(j * L, L)
            dst[r, sl] = (dst[r, sl] + src[r, sl]) * 0.5
        return c
    lax.fori_loop(0, rows, row_body, 0, unroll=False)


def _sc_kernel(xf, p1, p2, out, idx1_v, idx2_v,
               b1a, b2a, b1b, b2b, gsa, gsb, ssa, ssb, csem):
    w = lax.axis_index("s") * NC + lax.axis_index("c")
    b = w // WPB
    part = w % WPB
    boff = (b * V).astype(jnp.int32)

    coff = jnp.minimum(part * COPY_W, COPY_LAST)
    copy = pltpu.make_async_copy(
        xf.at[pl.ds(b * V + coff, COPY_W)],
        out.at[pl.ds(b * VOUT + coff, COPY_W)], csem)
    copy.start()

    n0 = part * N_PER_W
    orow0 = b * VOUT + V + n0

    pltpu.sync_copy(p1.at[pl.ds(n0, N_PER_W)], idx1_v.at[pl.ds(0, N_PER_W)])
    pltpu.sync_copy(p2.at[pl.ds(n0, N_PER_W)], idx2_v.at[pl.ds(0, N_PER_W)])

    def add_body(i, c):
        sl = pl.ds(i * L, L)
        idx1_v[sl] = idx1_v[sl] + boff
        idx2_v[sl] = idx2_v[sl] + boff
        return c
    lax.fori_loop(0, IDX_PAD // L, add_body, 0, unroll=False)

    def goff(g):
        return jnp.minimum(g * CHUNK, LAST_OFF)

    def start_gathers(g, d1, d2, sem):
        off = goff(g)
        pltpu.make_async_copy(
            xf.at[idx1_v.at[pl.ds(off, CHUNK)]], d1, sem).start()
        pltpu.make_async_copy(
            xf.at[idx2_v.at[pl.ds(off, CHUNK)]], d2, sem).start()

    def drain(dst, sem):
        pltpu.make_async_copy(xf.at[pl.ds(0, CHUNK)], dst, sem).wait()

    def drain_store(sem):
        pltpu.make_async_copy(
            b1a, out.at[pl.ds(orow0, CHUNK)], sem).wait()

    start_gathers(0, b1a, b2a, gsa)
    start_gathers(1, b1b, b2b, gsb)

    slots = ((b1a, b2a, gsa, ssa), (b1b, b2b, gsb, ssb))

    def pair_body(t, c):
        for s, (d1, d2, gsem, ssem) in enumerate(slots):
            g = 2 * t + s
            drain(d1, gsem)
            drain(d2, gsem)
            _avg_into(d1, d2, CHUNK)
            pltpu.make_async_copy(
                d1, out.at[pl.ds(orow0 + goff(g), CHUNK)], ssem).start()

            @pl.when(g + 2 < NCH)
            def _():
                drain_store(ssem)
                start_gathers(g + 2, d1, d2, gsem)
        return c

    lax.fori_loop(0, NCH // 2, pair_body, 0, unroll=False)
    drain_store(ssa)
    drain_store(ssb)
    copy.wait()


@jax.jit
def _unpool(xf, p1, p2):
    mesh = plsc.VectorSubcoreMesh(core_axis_name="c", subcore_axis_name="s")
    f = pl.kernel(
        _sc_kernel,
        out_type=jax.ShapeDtypeStruct((B * VOUT, F), jnp.float32),
        mesh=mesh,
        scratch_types=[
            pltpu.VMEM((IDX_PAD,), jnp.int32),
            pltpu.VMEM((IDX_PAD,), jnp.int32),
            pltpu.VMEM((CHUNK, F), jnp.float32),
            pltpu.VMEM((CHUNK, F), jnp.float32),
            pltpu.VMEM((CHUNK, F), jnp.float32),
            pltpu.VMEM((CHUNK, F), jnp.float32),
            pltpu.SemaphoreType.DMA,
            pltpu.SemaphoreType.DMA,
            pltpu.SemaphoreType.DMA,
            pltpu.SemaphoreType.DMA,
            pltpu.SemaphoreType.DMA,
        ],
    )
    return f(xf, p1, p2)


def kernel(x, pool_x1, pool_x2):
    xf = x.reshape(B * V, F)
    out = _unpool(xf, pool_x1.astype(jnp.int32), pool_x2.astype(jnp.int32))
    return out.reshape(B, VOUT, F)


# speedup vs baseline: 9.5854x; 9.5854x over previous
import jax
import jax.numpy as jnp
from jax import lax
from jax.experimental import pallas as pl
from jax.experimental.pallas import tpu as pltpu
from jax.experimental.pallas import tpu_sc as plsc

B = 8
V = 10000
F = 256
NNEW = 20000
NC, NS, L = 2, 16, 16
NW = NC * NS
PER_W = (B * NNEW) // NW
WPB = NW // B
N_PER_W = NNEW // WPB
CHUNK = 112
NCH = 46
LAST_OFF = PER_W - CHUNK
IDX_PAD = 5008
VOUT = V + NNEW
COPY_W = 2504
COPY_LAST = V - COPY_W
NCC = 24
COPY_CLAST = COPY_W - CHUNK


def _avg_into(dst, src, rows):
    def row_body(r, c):
        for j in range(F // L):
            sl = pl.ds(j * L, L)
            dst[r, sl] = (dst[r, sl] + src[r, sl]) * 0.5
        return c
    lax.fori_loop(0, rows, row_body, 0, unroll=False)


def _sc_kernel(xf, p1, p2, out, idx1_v, idx2_v,
               b1a, b2a, b1b, b2b, gsa, gsb, ssa, ssb):
    w = lax.axis_index("s") * NC + lax.axis_index("c")
    b = w // WPB
    part = w % WPB
    boff = (b * V).astype(jnp.int32)

    n0 = part * N_PER_W
    orow0 = b * VOUT + V + n0

    pltpu.sync_copy(p1.at[pl.ds(n0, N_PER_W)], idx1_v.at[pl.ds(0, N_PER_W)])
    pltpu.sync_copy(p2.at[pl.ds(n0, N_PER_W)], idx2_v.at[pl.ds(0, N_PER_W)])

    def add_body(i, c):
        sl = pl.ds(i * L, L)
        idx1_v[sl] = idx1_v[sl] + boff
        idx2_v[sl] = idx2_v[sl] + boff
        return c
    lax.fori_loop(0, IDX_PAD // L, add_body, 0, unroll=False)

    def goff(g):
        return jnp.minimum(g * CHUNK, LAST_OFF)

    def start_gathers(g, d1, d2, sem):
        off = goff(g)
        pltpu.make_async_copy(
            xf.at[idx1_v.at[pl.ds(off, CHUNK)]], d1, sem).start()
        pltpu.make_async_copy(
            xf.at[idx2_v.at[pl.ds(off, CHUNK)]], d2, sem).start()

    def drain_in(dst, sem):
        pltpu.make_async_copy(xf.at[pl.ds(0, CHUNK)], dst, sem).wait()

    def drain_out(sem):
        pltpu.make_async_copy(b1a, out.at[pl.ds(orow0, CHUNK)], sem).wait()

    start_gathers(0, b1a, b2a, gsa)
    start_gathers(1, b1b, b2b, gsb)
    slots = ((b1a, b2a, gsa, ssa), (b1b, b2b, gsb, ssb))

    def pair_body(t, c):
        for s, (d1, d2, gsem, ssem) in enumerate(slots):
            g = 2 * t + s
            drain_in(d1, gsem)
            drain_in(d2, gsem)
            _avg_into(d1, d2, CHUNK)
            pltpu.make_async_copy(
                d1, out.at[pl.ds(orow0 + goff(g), CHUNK)], ssem).start()

            @pl.when(g + 2 < NCH)
            def _():
                drain_out(ssem)
                start_gathers(g + 2, d1, d2, gsem)
        return c

    lax.fori_loop(0, NCH // 2, pair_body, 0, unroll=False)
    drain_out(ssa)
    drain_out(ssb)

    coff = jnp.minimum(part * COPY_W, COPY_LAST)
    src0 = b * V + coff
    dst0 = b * VOUT + coff

    def koff(k):
        return jnp.minimum(k * CHUNK, COPY_CLAST)

    def start_cin(k, buf, sem):
        pltpu.make_async_copy(
            xf.at[pl.ds(src0 + koff(k), CHUNK)], buf, sem).start()

    start_cin(0, b1a, gsa)
    start_cin(1, b1b, gsb)
    cslots = ((b1a, gsa, ssa), (b1b, gsb, ssb))

    def copy_body(t, c):
        for s, (buf, gsem, ssem) in enumerate(cslots):
            k = 2 * t + s
            drain_in(buf, gsem)
            pltpu.make_async_copy(
                buf, out.at[pl.ds(dst0 + koff(k), CHUNK)], ssem).start()

            @pl.when(k + 2 < NCC)
            def _():
                drain_out(ssem)
                start_cin(k + 2, buf, gsem)
        return c

    lax.fori_loop(0, NCC // 2, copy_body, 0, unroll=False)
    drain_out(ssa)
    drain_out(ssb)


@jax.jit
def _unpool(xf, p1, p2):
    mesh = plsc.VectorSubcoreMesh(core_axis_name="c", subcore_axis_name="s")
    f = pl.kernel(
        _sc_kernel,
        out_type=jax.ShapeDtypeStruct((B * VOUT, F), jnp.float32),
        mesh=mesh,
        scratch_types=[
            pltpu.VMEM((IDX_PAD,), jnp.int32),
            pltpu.VMEM((IDX_PAD,), jnp.int32),
            pltpu.VMEM((CHUNK, F), jnp.float32),
            pltpu.VMEM((CHUNK, F), jnp.float32),
            pltpu.VMEM((CHUNK, F), jnp.float32),
            pltpu.VMEM((CHUNK, F), jnp.float32),
            pltpu.SemaphoreType.DMA,
            pltpu.SemaphoreType.DMA,
            pltpu.SemaphoreType.DMA,
            pltpu.SemaphoreType.DMA,
        ],
    )
    return f(xf, p1, p2)


def kernel(x, pool_x1, pool_x2):
    xf = x.reshape(B * V, F)
    out = _unpool(xf, pool_x1.astype(jnp.int32), pool_x2.astype(jnp.int32))
    return out.reshape(B, VOUT, F)


# speedup vs baseline: 9.7081x; 1.0128x over previous
import jax
import jax.numpy as jnp
from jax import lax
from jax.experimental import pallas as pl
from jax.experimental.pallas import tpu as pltpu
from jax.experimental.pallas import tpu_sc as plsc

B = 8
V = 10000
F = 256
NNEW = 20000
NC, NS, L = 2, 16, 16
NW = NC * NS
PER_W = (B * NNEW) // NW
WPB = NW // B
N_PER_W = NNEW // WPB
CHUNK = 88
NCH = 58
LAST_OFF = PER_W - CHUNK
IDX_PAD = 5008
VOUT = V + NNEW
COPY_W = 2504
COPY_LAST = V - COPY_W
CCH = 48
NCC = 54
COPY_CLAST = COPY_W - CCH


def _avg_into(dst, src, rows):
    @plsc.parallel_loop(0, rows, step=1, unroll=2)
    def _(r):
        for j in range(F // L):
            sl = pl.ds(j * L, L)
            dst[r, sl] = (dst[r, sl] + src[r, sl]) * 0.5


def _sc_kernel(xf, p1, p2, out, idx1_v, idx2_v,
               b1a, b2a, b1b, b2b, cba, cbb,
               gsa, gsb, ssa, ssb, cia, cib, coa, cob):
    w = lax.axis_index("s") * NC + lax.axis_index("c")
    b = w // WPB
    part = w % WPB
    boff = (b * V).astype(jnp.int32)

    n0 = part * N_PER_W
    orow0 = b * VOUT + V + n0

    pltpu.sync_copy(p1.at[pl.ds(n0, N_PER_W)], idx1_v.at[pl.ds(0, N_PER_W)])
    pltpu.sync_copy(p2.at[pl.ds(n0, N_PER_W)], idx2_v.at[pl.ds(0, N_PER_W)])

    def add_body(i, c):
        sl = pl.ds(i * L, L)
        idx1_v[sl] = idx1_v[sl] + boff
        idx2_v[sl] = idx2_v[sl] + boff
        return c
    lax.fori_loop(0, IDX_PAD // L, add_body, 0, unroll=False)

    coff = jnp.minimum(part * COPY_W, COPY_LAST)
    src0 = b * V + coff
    dst0 = b * VOUT + coff

    def goff(g):
        return jnp.minimum(g * CHUNK, LAST_OFF)

    def koff(k):
        return jnp.minimum(k * CCH, COPY_CLAST)

    def start_gathers(g, d1, d2, sem):
        off = goff(g)
        pltpu.make_async_copy(
            xf.at[idx1_v.at[pl.ds(off, CHUNK)]], d1, sem).start()
        pltpu.make_async_copy(
            xf.at[idx2_v.at[pl.ds(off, CHUNK)]], d2, sem).start()

    def start_cin(k, buf, sem):
        pltpu.make_async_copy(
            xf.at[pl.ds(src0 + koff(k), CCH)], buf, sem).start()

    def drain_in(dst, sem):
        pltpu.make_async_copy(xf.at[pl.ds(0, CHUNK)], dst, sem).wait()

    def drain_cin(dst, sem):
        pltpu.make_async_copy(xf.at[pl.ds(0, CCH)], dst, sem).wait()

    def drain_store(sem):
        pltpu.make_async_copy(b1a, out.at[pl.ds(orow0, CHUNK)], sem).wait()

    def drain_cout(sem):
        pltpu.make_async_copy(cba, out.at[pl.ds(dst0, CCH)], sem).wait()

    start_gathers(0, b1a, b2a, gsa)
    start_gathers(1, b1b, b2b, gsb)
    start_cin(0, cba, cia)
    start_cin(1, cbb, cib)

    gslots = ((b1a, b2a, gsa, ssa), (b1b, b2b, gsb, ssb))
    cslots = ((cba, cia, coa), (cbb, cib, cob))

    def pair_body(t, c):
        for s, (d1, d2, gsem, ssem) in enumerate(gslots):
            g = 2 * t + s
            drain_in(d1, gsem)
            drain_in(d2, gsem)
            _avg_into(d1, d2, CHUNK)
            pltpu.make_async_copy(
                d1, out.at[pl.ds(orow0 + goff(g), CHUNK)], ssem).start()

            @pl.when(g + 2 < NCH)
            def _():
                drain_store(ssem)
                start_gathers(g + 2, d1, d2, gsem)

        for s, (buf, cisem, cosem) in enumerate(cslots):
            k = 2 * t + s

            @pl.when(k < NCC)
            def _():
                drain_cin(buf, cisem)
                pltpu.make_async_copy(
                    buf, out.at[pl.ds(dst0 + koff(k), CCH)], cosem).start()

                @pl.when(k + 2 < NCC)
                def _():
                    drain_cout(cosem)
                    start_cin(k + 2, buf, cisem)
        return c

    lax.fori_loop(0, NCH // 2, pair_body, 0, unroll=False)
    drain_store(ssa)
    drain_store(ssb)
    drain_cout(coa)
    drain_cout(cob)


@jax.jit
def _unpool(xf, p1, p2):
    mesh = plsc.VectorSubcoreMesh(core_axis_name="c", subcore_axis_name="s")
    f = pl.kernel(
        _sc_kernel,
        out_type=jax.ShapeDtypeStruct((B * VOUT, F), jnp.float32),
        mesh=mesh,
        scratch_types=[
            pltpu.VMEM((IDX_PAD,), jnp.int32),
            pltpu.VMEM((IDX_PAD,), jnp.int32),
            pltpu.VMEM((CHUNK, F), jnp.float32),
            pltpu.VMEM((CHUNK, F), jnp.float32),
            pltpu.VMEM((CHUNK, F), jnp.float32),
            pltpu.VMEM((CHUNK, F), jnp.float32),
            pltpu.VMEM((CCH, F), jnp.float32),
            pltpu.VMEM((CCH, F), jnp.float32),
            pltpu.SemaphoreType.DMA,
            pltpu.SemaphoreType.DMA,
            pltpu.SemaphoreType.DMA,
            pltpu.SemaphoreType.DMA,
            pltpu.SemaphoreType.DMA,
            pltpu.SemaphoreType.DMA,
            pltpu.SemaphoreType.DMA,
            pltpu.SemaphoreType.DMA,
        ],
    )
    return f(xf, p1, p2)


def kernel(x, pool_x1, pool_x2):
    xf = x.reshape(B * V, F)
    out = _unpool(xf, pool_x1.astype(jnp.int32), pool_x2.astype(jnp.int32))
    return out.reshape(B, VOUT, F)


# speedup vs baseline: 9.7220x; 1.0014x over previous
import jax
import jax.numpy as jnp
from jax import lax
from jax.experimental import pallas as pl
from jax.experimental.pallas import tpu as pltpu
from jax.experimental.pallas import tpu_sc as plsc

B = 8
V = 10000
F = 256
NNEW = 20000
NC, NS, L = 2, 16, 16
NW = NC * NS
PER_W = (B * NNEW) // NW
WPB = NW // B
N_PER_W = NNEW // WPB
CHUNK = 88
NCH = 58
LAST_OFF = PER_W - CHUNK
IDX_PAD = 5008
VOUT = V + NNEW
COPY_W = 2504
COPY_LAST = V - COPY_W
CCH = 48
NCC = 54
COPY_CLAST = COPY_W - CCH


def _avg_into(dst, src, rows):
    @plsc.parallel_loop(0, rows, step=1, unroll=2)
    def _(r):
        for j in range(F // L):
            sl = pl.ds(j * L, L)
            dst[r, sl] = (dst[r, sl] + src[r, sl]) * 0.5


def _sc_kernel(xf, p1, p2, out, idx1_v, idx2_v,
               b1a, b2a, b1b, b2b, cba, cbb,
               gsa, gsb, ssa, ssb, cia, cib, coa, cob):
    w = lax.axis_index("s") * NC + lax.axis_index("c")
    b = w // WPB
    part = w % WPB
    boff = (b * V).astype(jnp.int32)

    n0 = part * N_PER_W
    orow0 = b * VOUT + V + n0

    pltpu.sync_copy(p1.at[pl.ds(n0, N_PER_W)], idx1_v.at[pl.ds(0, N_PER_W)])
    pltpu.sync_copy(p2.at[pl.ds(n0, N_PER_W)], idx2_v.at[pl.ds(0, N_PER_W)])

    def add_body(i, c):
        sl = pl.ds(i * L, L)
        idx1_v[sl] = idx1_v[sl] + boff
        idx2_v[sl] = idx2_v[sl] + boff
        return c
    lax.fori_loop(0, IDX_PAD // L, add_body, 0, unroll=False)

    coff = jnp.minimum(part * COPY_W, COPY_LAST)
    src0 = b * V + coff
    dst0 = b * VOUT + coff

    def goff(g):
        return jnp.minimum(g * CHUNK, LAST_OFF)

    def koff(k):
        return jnp.minimum(k * CCH, COPY_CLAST)

    def start_gather(g, idx_v, dst, sem):
        pltpu.make_async_copy(
            xf.at[idx_v.at[pl.ds(goff(g), CHUNK)]], dst, sem).start()

    def start_cin(k, buf, sem):
        pltpu.make_async_copy(
            xf.at[pl.ds(src0 + koff(k), CCH)], buf, sem).start()

    def drain_in(dst, sem):
        pltpu.make_async_copy(xf.at[pl.ds(0, CHUNK)], dst, sem).wait()

    def drain_cin(dst, sem):
        pltpu.make_async_copy(xf.at[pl.ds(0, CCH)], dst, sem).wait()

    def drain_store(sem):
        pltpu.make_async_copy(b1a, out.at[pl.ds(orow0, CHUNK)], sem).wait()

    def drain_cout(sem):
        pltpu.make_async_copy(cba, out.at[pl.ds(dst0, CCH)], sem).wait()

    start_gather(0, idx1_v, b1a, gsa)
    start_gather(0, idx2_v, b2a, gsa)
    start_gather(1, idx1_v, b1b, gsb)
    start_gather(1, idx2_v, b2b, gsb)
    start_cin(0, cba, cia)
    start_cin(1, cbb, cib)

    gslots = ((b1a, b2a, gsa, ssa), (b1b, b2b, gsb, ssb))
    cslots = ((cba, cia, coa), (cbb, cib, cob))

    def pair_body(t, c):
        for s, (d1, d2, gsem, ssem) in enumerate(gslots):
            g = 2 * t + s
            drain_in(d1, gsem)
            drain_in(d2, gsem)
            _avg_into(d1, d2, CHUNK)
            pltpu.make_async_copy(
                d1, out.at[pl.ds(orow0 + goff(g), CHUNK)], ssem).start()

            @pl.when(g + 2 < NCH)
            def _():
                start_gather(g + 2, idx2_v, d2, gsem)
                drain_store(ssem)
                start_gather(g + 2, idx1_v, d1, gsem)

        for s, (buf, cisem, cosem) in enumerate(cslots):
            k = 2 * t + s

            @pl.when(k < NCC)
            def _():
                drain_cin(buf, cisem)
                pltpu.make_async_copy(
                    buf, out.at[pl.ds(dst0 + koff(k), CCH)], cosem).start()

                @pl.when(k + 2 < NCC)
                def _():
                    drain_cout(cosem)
                    start_cin(k + 2, buf, cisem)
        return c

    lax.fori_loop(0, NCH // 2, pair_body, 0, unroll=False)
    drain_store(ssa)
    drain_store(ssb)
    drain_cout(coa)
    drain_cout(cob)


@jax.jit
def _unpool(xf, p1, p2):
    mesh = plsc.VectorSubcoreMesh(core_axis_name="c", subcore_axis_name="s")
    f = pl.kernel(
        _sc_kernel,
        out_type=jax.ShapeDtypeStruct((B * VOUT, F), jnp.float32),
        mesh=mesh,
        scratch_types=[
            pltpu.VMEM((IDX_PAD,), jnp.int32),
            pltpu.VMEM((IDX_PAD,), jnp.int32),
            pltpu.VMEM((CHUNK, F), jnp.float32),
            pltpu.VMEM((CHUNK, F), jnp.float32),
            pltpu.VMEM((CHUNK, F), jnp.float32),
            pltpu.VMEM((CHUNK, F), jnp.float32),
            pltpu.VMEM((CCH, F), jnp.float32),
            pltpu.VMEM((CCH, F), jnp.float32),
            pltpu.SemaphoreType.DMA,
            pltpu.SemaphoreType.DMA,
            pltpu.SemaphoreType.DMA,
            pltpu.SemaphoreType.DMA,
            pltpu.SemaphoreType.DMA,
            pltpu.SemaphoreType.DMA,
            pltpu.SemaphoreType.DMA,
            pltpu.SemaphoreType.DMA,
        ],
    )
    return f(xf, p1, p2)


def kernel(x, pool_x1, pool_x2):
    xf = x.reshape(B * V, F)
    out = _unpool(xf, pool_x1.astype(jnp.int32), pool_x2.astype(jnp.int32))
    return out.reshape(B, VOUT, F)
